# Initial kernel scaffold; baseline (speedup 1.0000x reference)
import jax
import jax.numpy as jnp
from jax.experimental import pallas as pl


def kernel(item_ids, embeddings, H0, H1, H2, H3, W, b):
    raise NotImplementedError("write your pallas kernel here")


# speedup vs baseline: 35.3640x; 35.3640x over previous
import functools

import jax
import jax.numpy as jnp
from jax import lax
from jax.experimental import pallas as pl
from jax.experimental.pallas import tpu as pltpu
from jax.experimental.pallas import tpu_sc as plsc

BATCH = 4096
SEQ = 200
DIM = 64
NB = 1024

_NC = 2
_NS = 16
_L = 16
_NW = _NC * _NS
_ROWS_W = BATCH // _NW
_R = _L
_CHUNKS = _ROWS_W // _R


def _sc_hist_body(ids_hbm, out_hbm, ids_v, cnt_v):
    wid = lax.axis_index("s") * _NC + lax.axis_index("c")
    row0 = wid * _ROWS_W

    zeros16 = jnp.zeros((_L,), jnp.float32)
    ones16 = jnp.ones((_L,), jnp.float32)
    lane = lax.iota(jnp.int32, _L)
    lane_seq = lane * SEQ
    lane_nb = lane * NB

    def chunk_body(ci, _):
        base = row0 + ci * _R
        pltpu.sync_copy(ids_hbm.at[pl.ds(base * SEQ, _R * SEQ)], ids_v)

        def zbody(j, _):
            for u in range(16):
                cnt_v[pl.ds((j * 16 + u) * _L, _L)] = zeros16
            return 0

        lax.fori_loop(0, _R * NB // (_L * 16), zbody, 0, unroll=False)

        def sbody(s, _):
            for u in range(4):
                ids16 = plsc.load_gather(ids_v, [lane_seq + (s * 4 + u)])
                tgt = lane_nb + jnp.bitwise_and(ids16, NB - 1)
                plsc.addupdate_scatter(cnt_v, [tgt], ones16)
            return 0

        lax.fori_loop(0, SEQ // 4, sbody, 0, unroll=False)

        pltpu.sync_copy(cnt_v, out_hbm.at[pl.ds(base * NB, _R * NB)])
        return 0

    lax.fori_loop(0, _CHUNKS, chunk_body, 0, unroll=False)


@jax.jit
def _sc_histogram(ids_flat):
    mesh = plsc.VectorSubcoreMesh(core_axis_name="c", subcore_axis_name="s")
    f = pl.kernel(
        _sc_hist_body,
        out_type=jax.ShapeDtypeStruct((BATCH * NB,), jnp.float32),
        mesh=mesh,
        scratch_types=[
            pltpu.VMEM((_R * SEQ,), jnp.int32),
            pltpu.VMEM((_R * NB,), jnp.float32),
        ],
        compiler_params=pltpu.CompilerParams(needs_layout_passes=False),
    )
    return f(ids_flat)


_BB = 256


def _tc_body(emb_ref, cnt_ref, hcat_ref, w_ref, b_ref, out_ref):
    t = jnp.sum(emb_ref[...], axis=1)
    esum = t[:, :DIM] + t[:, DIM:]
    hsum = jnp.dot(cnt_ref[...], hcat_ref[...],
                   preferred_element_type=jnp.float32)
    comb = (esum + hsum) * (1.0 / SEQ)
    proj = lax.dot_general(comb, w_ref[...], (((1,), (1,)), ((), ())),
                           preferred_element_type=jnp.float32)
    out_ref[...] = proj + b_ref[...]


@jax.jit
def _tc_combine(emb3, counts, hcat, w, b2):
    grid = (BATCH // _BB,)
    return pl.pallas_call(
        _tc_body,
        grid=grid,
        in_specs=[
            pl.BlockSpec((_BB, SEQ * DIM // 128, 128), lambda i: (i, 0, 0)),
            pl.BlockSpec((_BB, NB), lambda i: (i, 0)),
            pl.BlockSpec((NB, DIM), lambda i: (0, 0)),
            pl.BlockSpec((DIM, DIM), lambda i: (0, 0)),
            pl.BlockSpec((1, DIM), lambda i: (0, 0)),
        ],
        out_specs=pl.BlockSpec((_BB, DIM), lambda i: (i, 0)),
        out_shape=jax.ShapeDtypeStruct((BATCH, DIM), jnp.float32),
    )(emb3, counts, hcat, w, b2)


def kernel(item_ids, embeddings, H0, H1, H2, H3, W, b):
    ids_flat = item_ids.reshape(-1).astype(jnp.int32)
    counts = _sc_histogram(ids_flat).reshape(BATCH, NB)
    hcat = jnp.concatenate([H0, H1, H2, H3], axis=1)
    emb3 = embeddings.reshape(BATCH, SEQ * DIM // 128, 128)
    return _tc_combine(emb3, counts, hcat, W, b.reshape(1, DIM))
